# Initial kernel scaffold; baseline (speedup 1.0000x reference)
import jax
import jax.numpy as jnp
from jax.experimental import pallas as pl


def kernel(x, edge_index, Ws0, Wn0, b0, ln_g0, ln_b0, Ws1, Wn1, b1, ln_g1, ln_b1, W1, bc1, bn_g, bn_b, W2, bc2):
    raise NotImplementedError("write your pallas kernel here")


# speedup vs baseline: 1.1769x; 1.1769x over previous
import functools

import jax
import jax.numpy as jnp
from jax import lax
from jax.experimental import pallas as pl
from jax.experimental.pallas import tpu as pltpu
from jax.experimental.pallas import tpu_sc as plsc

N = 50000
R = 7
E = 80000
D = 128
H = 128
HC = 64
OUT = 2

NC = 2
NS = 16
NCH = 4
DC = 32
CNTW = 16

EP = E // NS
BATCH = 128
NBATCH = 40
EPP = NBATCH * BATCH
DUMP = 50040

NP = 50048
RPT = NP // NS
PIECE = 184
NPIECE = RPT // PIECE

_mesh = plsc.VectorSubcoreMesh(
    core_axis_name="c", subcore_axis_name="s", num_cores=NC, num_subcores=NS)


@functools.partial(
    pl.kernel,
    out_type=jax.ShapeDtypeStruct((R, NP, CNTW), jnp.float32),
    mesh=_mesh,
    compiler_params=pltpu.CompilerParams(use_tc_tiling_on_sc=False),
    scratch_types=dict(
        acc=pltpu.VMEM_SHARED((NP, CNTW), jnp.float32),
        ones=pltpu.VMEM((BATCH, CNTW), jnp.float32),
        didx=pltpu.VMEM((BATCH,), jnp.int32),
        zbuf=pltpu.VMEM((PIECE, CNTW), jnp.float32),
        wbuf=pltpu.VMEM((PIECE, CNTW), jnp.float32),
    ),
)
def _sc_counts(edst_hbm, zeros_hbm, ones_hbm, cnt_hbm, acc, ones, didx,
               zbuf, wbuf):
    cid = lax.axis_index("c")
    sid = lax.axis_index("s")
    pltpu.sync_copy(ones_hbm, ones)
    pltpu.sync_copy(zeros_hbm, zbuf)
    row0 = sid * RPT
    for r in range(R):
        mine = cid == (r % NC)

        @pl.when(mine)
        def _():
            def zstep(p, _):
                pltpu.sync_copy(zbuf, acc.at[pl.ds(row0 + p * PIECE, PIECE)])
                return 0

            lax.fori_loop(0, NPIECE, zstep, 0)

        plsc.subcore_barrier()

        @pl.when(mine)
        def _():
            base = (r * NS + sid) * EPP

            def bstep(b, _):
                pltpu.sync_copy(edst_hbm.at[pl.ds(base + b * BATCH, BATCH)],
                                didx)
                pltpu.sync_copy(ones, acc.at[didx], add=True)
                return 0

            lax.fori_loop(0, NBATCH, bstep, 0)

        plsc.subcore_barrier()

        @pl.when(mine)
        def _():
            def wstep(p, _):
                o = row0 + p * PIECE
                pltpu.sync_copy(acc.at[pl.ds(o, PIECE)], wbuf)
                pltpu.sync_copy(wbuf, cnt_hbm.at[r, pl.ds(o, PIECE)])
                return 0

            lax.fori_loop(0, NPIECE, wstep, 0)

        plsc.subcore_barrier()


@functools.partial(
    pl.kernel,
    out_type=jax.ShapeDtypeStruct((R, NCH, NP, DC), jnp.float32),
    mesh=_mesh,
    compiler_params=pltpu.CompilerParams(use_tc_tiling_on_sc=False),
    scratch_types=dict(
        acc=pltpu.VMEM_SHARED((NP, DC), jnp.float32),
        sidx=pltpu.VMEM((BATCH,), jnp.int32),
        didx=pltpu.VMEM((BATCH,), jnp.int32),
        rows=pltpu.VMEM((BATCH, DC), jnp.float32),
        zbuf=pltpu.VMEM((PIECE, DC), jnp.float32),
        wbuf=pltpu.VMEM((PIECE, DC), jnp.float32),
        sem=pltpu.SemaphoreType.DMA,
    ),
)
def _sc_msg(h0, h1, h2, h3, esrc_hbm, edst_hbm, zeros_hbm, msg_hbm, acc,
            sidx, didx, rows, zbuf, wbuf, sem):
    cid = lax.axis_index("c")
    sid = lax.axis_index("s")
    pltpu.sync_copy(zeros_hbm, zbuf)
    row0 = sid * RPT
    hcs = [h0, h1, h2, h3]
    for r in range(R):
        for c in range(NCH):
            hc = hcs[c]
            t = r * NCH + c

            mine = cid == (t % NC)

            @pl.when(mine)
            def _():
                def zstep(p, _):
                    pltpu.sync_copy(zbuf,
                                    acc.at[pl.ds(row0 + p * PIECE, PIECE)])
                    return 0

                lax.fori_loop(0, NPIECE, zstep, 0)

            plsc.subcore_barrier()

            @pl.when(mine)
            def _():
                base = (r * NS + sid) * EPP

                def bstep(b, _):
                    o = base + b * BATCH
                    pltpu.sync_copy(esrc_hbm.at[pl.ds(o, BATCH)], sidx)
                    pltpu.sync_copy(edst_hbm.at[pl.ds(o, BATCH)], didx)
                    pltpu.async_copy(hc.at[sidx], rows, sem).wait()
                    pltpu.sync_copy(rows, acc.at[didx], add=True)
                    return 0

                lax.fori_loop(0, NBATCH, bstep, 0)

            plsc.subcore_barrier()

            @pl.when(mine)
            def _():
                def wstep(p, _):
                    o = row0 + p * PIECE
                    pltpu.sync_copy(acc.at[pl.ds(o, PIECE)], wbuf)
                    pltpu.sync_copy(wbuf, msg_hbm.at[r, c, pl.ds(o, PIECE)])
                    return 0

                lax.fori_loop(0, NPIECE, wstep, 0)

            plsc.subcore_barrier()


BN = 1000


def _layer_body(h_ref, msg_ref, cnt_ref, Ws_ref, Wn_ref, b_ref, g_ref, be_ref,
                out_ref, outc_ref):
    h = h_ref[...]
    Ws_avg = jnp.mean(Ws_ref[...], axis=0)
    acc = jnp.dot(h, Ws_avg, preferred_element_type=jnp.float32)
    acc = acc + jnp.mean(b_ref[...], axis=0)[None, :]
    for r in range(R):
        inv = 1.0 / (R * jnp.maximum(cnt_ref[r, :, 0], 1.0))
        m = jnp.concatenate([msg_ref[r, c] for c in range(NCH)], axis=-1)
        acc = acc + jnp.dot(m * inv[:, None], Wn_ref[r],
                            preferred_element_type=jnp.float32)
    mu = jnp.mean(acc, axis=-1, keepdims=True)
    var = jnp.mean((acc - mu) * (acc - mu), axis=-1, keepdims=True)
    hn = (acc - mu) * lax.rsqrt(var + 1e-5)
    hn = hn * g_ref[...] + be_ref[...]
    hn = jnp.maximum(hn, 0.0)
    out_ref[...] = hn
    for c in range(NCH):
        outc_ref[c] = hn[:, c * DC:(c + 1) * DC]


def _layer_tc(h, msg, cnt, Ws, Wn, b, g, be):
    g2 = g.reshape(1, H)
    be2 = be.reshape(1, H)
    return pl.pallas_call(
        _layer_body,
        grid=(N // BN,),
        in_specs=[
            pl.BlockSpec((BN, H), lambda i: (i, 0)),
            pl.BlockSpec((R, NCH, BN, DC), lambda i: (0, 0, i, 0)),
            pl.BlockSpec((R, BN, CNTW), lambda i: (0, i, 0)),
            pl.BlockSpec((R, D, H), lambda i: (0, 0, 0)),
            pl.BlockSpec((R, D, H), lambda i: (0, 0, 0)),
            pl.BlockSpec((R, H), lambda i: (0, 0)),
            pl.BlockSpec((1, H), lambda i: (0, 0)),
            pl.BlockSpec((1, H), lambda i: (0, 0)),
        ],
        out_specs=[
            pl.BlockSpec((BN, H), lambda i: (i, 0)),
            pl.BlockSpec((NCH, BN, DC), lambda i: (0, i, 0)),
        ],
        out_shape=[
            jax.ShapeDtypeStruct((N, H), jnp.float32),
            jax.ShapeDtypeStruct((NCH, N, DC), jnp.float32),
        ],
    )(h, msg, cnt, Ws, Wn, b, g2, be2)


def _clf_body(h_ref, W1_ref, b1_ref, g_ref, be_ref, W2_ref, b2_ref, out_ref):
    t = jnp.dot(h_ref[...], W1_ref[...], preferred_element_type=jnp.float32)
    t = t + b1_ref[...]
    t = jnp.maximum(t, 0.0)
    t = t / jnp.sqrt(1.0 + 1e-5) * g_ref[...] + be_ref[...]
    out_ref[...] = jnp.dot(t, W2_ref[...],
                           preferred_element_type=jnp.float32) + b2_ref[...]


def _clf_tc(h, W1, b1, g, be, W2, b2):
    return pl.pallas_call(
        _clf_body,
        grid=(N // BN,),
        in_specs=[
            pl.BlockSpec((BN, H), lambda i: (i, 0)),
            pl.BlockSpec((H, HC), lambda i: (0, 0)),
            pl.BlockSpec((1, HC), lambda i: (0, 0)),
            pl.BlockSpec((1, HC), lambda i: (0, 0)),
            pl.BlockSpec((1, HC), lambda i: (0, 0)),
            pl.BlockSpec((HC, OUT), lambda i: (0, 0)),
            pl.BlockSpec((1, OUT), lambda i: (0, 0)),
        ],
        out_specs=pl.BlockSpec((BN, OUT), lambda i: (i, 0)),
        out_shape=jax.ShapeDtypeStruct((N, OUT), jnp.float32),
    )(h, W1, b1.reshape(1, HC), g.reshape(1, HC), be.reshape(1, HC),
      W2, b2.reshape(1, OUT))


def kernel(x, edge_index, Ws0, Wn0, b0, ln_g0, ln_b0, Ws1, Wn1, b1, ln_g1,
           ln_b1, W1, bc1, bn_g, bn_b, W2, bc2):
    zeros32 = jnp.zeros((PIECE, DC), jnp.float32)
    zeros16 = jnp.zeros((PIECE, CNTW), jnp.float32)
    ones16 = jnp.ones((BATCH, CNTW), jnp.float32)

    et = edge_index.reshape(R, 2, NS, EP)
    pad = [(0, 0)] * 2 + [(0, EPP - EP)]
    esrc = jnp.pad(et[:, 0], pad, constant_values=0).reshape(-1)
    edst = jnp.pad(et[:, 1], pad, constant_values=DUMP).reshape(-1)

    cnt = _sc_counts(edst, zeros16, ones16)

    xc = jnp.transpose(x.reshape(N, NCH, DC), (1, 0, 2))
    msg0 = _sc_msg(xc[0], xc[1], xc[2], xc[3], esrc, edst, zeros32)
    h1, h1c = _layer_tc(x, msg0, cnt, Ws0, Wn0, b0, ln_g0, ln_b0)

    msg1 = _sc_msg(h1c[0], h1c[1], h1c[2], h1c[3], esrc, edst, zeros32)
    h2, _ = _layer_tc(h1, msg1, cnt, Ws1, Wn1, b1, ln_g1, ln_b1)

    return _clf_tc(h2, W1, bc1, bn_g, bn_b, W2, bc2)


# speedup vs baseline: 1.3085x; 1.1118x over previous
import functools

import jax
import jax.numpy as jnp
from jax import lax
from jax.experimental import pallas as pl
from jax.experimental.pallas import tpu as pltpu
from jax.experimental.pallas import tpu_sc as plsc

N = 50000
R = 7
E = 80000
D = 128
H = 128
HC = 64
OUT = 2

NC = 2
NS = 16
NCH = 4
DC = 32
CNTW = 16

EP = E // NS
BATCH = 128
NBATCH = 40
EPP = NBATCH * BATCH
DUMP = 50040

NP = 50048
RPT = NP // NS
PIECE = 184
NPIECE = RPT // PIECE

_mesh = plsc.VectorSubcoreMesh(
    core_axis_name="c", subcore_axis_name="s", num_cores=NC, num_subcores=NS)


@functools.partial(
    pl.kernel,
    out_type=jax.ShapeDtypeStruct((R, NP, CNTW), jnp.float32),
    mesh=_mesh,
    compiler_params=pltpu.CompilerParams(use_tc_tiling_on_sc=False),
    scratch_types=dict(
        acc=pltpu.VMEM_SHARED((NP, CNTW), jnp.float32),
        ones=pltpu.VMEM((BATCH, CNTW), jnp.float32),
        didx=pltpu.VMEM((BATCH,), jnp.int32),
        zbuf=pltpu.VMEM((PIECE, CNTW), jnp.float32),
        wbuf=pltpu.VMEM((PIECE, CNTW), jnp.float32),
    ),
)
def _sc_counts(edst_hbm, zeros_hbm, ones_hbm, cnt_hbm, acc, ones, didx,
               zbuf, wbuf):
    cid = lax.axis_index("c")
    sid = lax.axis_index("s")
    pltpu.sync_copy(ones_hbm, ones)
    pltpu.sync_copy(zeros_hbm, zbuf)
    row0 = sid * RPT
    for r in range(R):
        mine = cid == (r % NC)

        @pl.when(mine)
        def _():
            def zstep(p, _):
                pltpu.sync_copy(zbuf, acc.at[pl.ds(row0 + p * PIECE, PIECE)])
                return 0

            lax.fori_loop(0, NPIECE, zstep, 0)

        plsc.subcore_barrier()

        @pl.when(mine)
        def _():
            base = (r * NS + sid) * EPP

            def bstep(b, _):
                pltpu.sync_copy(edst_hbm.at[pl.ds(base + b * BATCH, BATCH)],
                                didx)
                pltpu.sync_copy(ones, acc.at[didx], add=True)
                return 0

            lax.fori_loop(0, NBATCH, bstep, 0)

        plsc.subcore_barrier()

        @pl.when(mine)
        def _():
            def wstep(p, _):
                o = row0 + p * PIECE
                pltpu.sync_copy(acc.at[pl.ds(o, PIECE)], wbuf)
                pltpu.sync_copy(wbuf, cnt_hbm.at[r, pl.ds(o, PIECE)])
                return 0

            lax.fori_loop(0, NPIECE, wstep, 0)

        plsc.subcore_barrier()


@functools.partial(
    pl.kernel,
    out_type=jax.ShapeDtypeStruct((R, NCH, NP, DC), jnp.float32),
    mesh=_mesh,
    compiler_params=pltpu.CompilerParams(use_tc_tiling_on_sc=False),
    scratch_types=dict(
        acc=pltpu.VMEM_SHARED((NP, DC), jnp.float32),
        sidx=pltpu.VMEM((BATCH,), jnp.int32),
        didx=pltpu.VMEM((BATCH,), jnp.int32),
        rows=pltpu.VMEM((BATCH, DC), jnp.float32),
        sidx2=pltpu.VMEM((BATCH,), jnp.int32),
        didx2=pltpu.VMEM((BATCH,), jnp.int32),
        rows2=pltpu.VMEM((BATCH, DC), jnp.float32),
        zbuf=pltpu.VMEM((PIECE, DC), jnp.float32),
        wbuf=pltpu.VMEM((PIECE, DC), jnp.float32),
        sem=pltpu.SemaphoreType.DMA,
        sem2=pltpu.SemaphoreType.DMA,
    ),
)
def _sc_msg(h0, h1, h2, h3, esrc_hbm, edst_hbm, zeros_hbm, msg_hbm, acc,
            sidx, didx, rows, sidx2, didx2, rows2, zbuf, wbuf, sem, sem2):
    cid = lax.axis_index("c")
    sid = lax.axis_index("s")
    pltpu.sync_copy(zeros_hbm, zbuf)
    row0 = sid * RPT
    hcs = [h0, h1, h2, h3]
    for r in range(R):
        for c in range(NCH):
            hc = hcs[c]
            t = r * NCH + c

            mine = cid == (t % NC)

            @pl.when(mine)
            def _():
                def zstep(p, _):
                    pltpu.sync_copy(zbuf,
                                    acc.at[pl.ds(row0 + p * PIECE, PIECE)])
                    return 0

                lax.fori_loop(0, NPIECE, zstep, 0)

            plsc.subcore_barrier()

            @pl.when(mine)
            def _():
                base = (r * NS + sid) * EPP

                def bstep(i, _):
                    o = base + (2 * i) * BATCH
                    o2 = o + BATCH
                    pltpu.sync_copy(esrc_hbm.at[pl.ds(o, BATCH)], sidx)
                    pltpu.sync_copy(edst_hbm.at[pl.ds(o, BATCH)], didx)
                    g = pltpu.async_copy(hc.at[sidx], rows, sem)
                    pltpu.sync_copy(esrc_hbm.at[pl.ds(o2, BATCH)], sidx2)
                    pltpu.sync_copy(edst_hbm.at[pl.ds(o2, BATCH)], didx2)
                    g2 = pltpu.async_copy(hc.at[sidx2], rows2, sem2)
                    g.wait()
                    pltpu.sync_copy(rows, acc.at[didx], add=True)
                    g2.wait()
                    pltpu.sync_copy(rows2, acc.at[didx2], add=True)
                    return 0

                lax.fori_loop(0, NBATCH // 2, bstep, 0)

            plsc.subcore_barrier()

            @pl.when(mine)
            def _():
                def wstep(p, _):
                    o = row0 + p * PIECE
                    pltpu.sync_copy(acc.at[pl.ds(o, PIECE)], wbuf)
                    pltpu.sync_copy(wbuf, msg_hbm.at[r, c, pl.ds(o, PIECE)])
                    return 0

                lax.fori_loop(0, NPIECE, wstep, 0)

            plsc.subcore_barrier()


BN = 1000


def _layer_body(h_ref, msg_ref, cnt_ref, Ws_ref, Wn_ref, b_ref, g_ref, be_ref,
                out_ref, outc_ref):
    h = h_ref[...]
    Ws_avg = jnp.mean(Ws_ref[...], axis=0)
    acc = jnp.dot(h, Ws_avg, preferred_element_type=jnp.float32)
    acc = acc + jnp.mean(b_ref[...], axis=0)[None, :]
    for r in range(R):
        inv = 1.0 / (R * jnp.maximum(cnt_ref[r, :, 0], 1.0))
        m = jnp.concatenate([msg_ref[r, c] for c in range(NCH)], axis=-1)
        acc = acc + jnp.dot(m * inv[:, None], Wn_ref[r],
                            preferred_element_type=jnp.float32)
    mu = jnp.mean(acc, axis=-1, keepdims=True)
    var = jnp.mean((acc - mu) * (acc - mu), axis=-1, keepdims=True)
    hn = (acc - mu) * lax.rsqrt(var + 1e-5)
    hn = hn * g_ref[...] + be_ref[...]
    hn = jnp.maximum(hn, 0.0)
    out_ref[...] = hn
    for c in range(NCH):
        outc_ref[c] = hn[:, c * DC:(c + 1) * DC]


def _layer_tc(h, msg, cnt, Ws, Wn, b, g, be):
    g2 = g.reshape(1, H)
    be2 = be.reshape(1, H)
    return pl.pallas_call(
        _layer_body,
        grid=(N // BN,),
        in_specs=[
            pl.BlockSpec((BN, H), lambda i: (i, 0)),
            pl.BlockSpec((R, NCH, BN, DC), lambda i: (0, 0, i, 0)),
            pl.BlockSpec((R, BN, CNTW), lambda i: (0, i, 0)),
            pl.BlockSpec((R, D, H), lambda i: (0, 0, 0)),
            pl.BlockSpec((R, D, H), lambda i: (0, 0, 0)),
            pl.BlockSpec((R, H), lambda i: (0, 0)),
            pl.BlockSpec((1, H), lambda i: (0, 0)),
            pl.BlockSpec((1, H), lambda i: (0, 0)),
        ],
        out_specs=[
            pl.BlockSpec((BN, H), lambda i: (i, 0)),
            pl.BlockSpec((NCH, BN, DC), lambda i: (0, i, 0)),
        ],
        out_shape=[
            jax.ShapeDtypeStruct((N, H), jnp.float32),
            jax.ShapeDtypeStruct((NCH, N, DC), jnp.float32),
        ],
    )(h, msg, cnt, Ws, Wn, b, g2, be2)


def _clf_body(h_ref, W1_ref, b1_ref, g_ref, be_ref, W2_ref, b2_ref, out_ref):
    t = jnp.dot(h_ref[...], W1_ref[...], preferred_element_type=jnp.float32)
    t = t + b1_ref[...]
    t = jnp.maximum(t, 0.0)
    t = t / jnp.sqrt(1.0 + 1e-5) * g_ref[...] + be_ref[...]
    out_ref[...] = jnp.dot(t, W2_ref[...],
                           preferred_element_type=jnp.float32) + b2_ref[...]


def _clf_tc(h, W1, b1, g, be, W2, b2):
    return pl.pallas_call(
        _clf_body,
        grid=(N // BN,),
        in_specs=[
            pl.BlockSpec((BN, H), lambda i: (i, 0)),
            pl.BlockSpec((H, HC), lambda i: (0, 0)),
            pl.BlockSpec((1, HC), lambda i: (0, 0)),
            pl.BlockSpec((1, HC), lambda i: (0, 0)),
            pl.BlockSpec((1, HC), lambda i: (0, 0)),
            pl.BlockSpec((HC, OUT), lambda i: (0, 0)),
            pl.BlockSpec((1, OUT), lambda i: (0, 0)),
        ],
        out_specs=pl.BlockSpec((BN, OUT), lambda i: (i, 0)),
        out_shape=jax.ShapeDtypeStruct((N, OUT), jnp.float32),
    )(h, W1, b1.reshape(1, HC), g.reshape(1, HC), be.reshape(1, HC),
      W2, b2.reshape(1, OUT))


def kernel(x, edge_index, Ws0, Wn0, b0, ln_g0, ln_b0, Ws1, Wn1, b1, ln_g1,
           ln_b1, W1, bc1, bn_g, bn_b, W2, bc2):
    zeros32 = jnp.zeros((PIECE, DC), jnp.float32)
    zeros16 = jnp.zeros((PIECE, CNTW), jnp.float32)
    ones16 = jnp.ones((BATCH, CNTW), jnp.float32)

    et = edge_index.reshape(R, 2, NS, EP)
    pad = [(0, 0)] * 2 + [(0, EPP - EP)]
    esrc = jnp.pad(et[:, 0], pad, constant_values=0).reshape(-1)
    edst = jnp.pad(et[:, 1], pad, constant_values=DUMP).reshape(-1)

    cnt = _sc_counts(edst, zeros16, ones16)

    xc = jnp.transpose(x.reshape(N, NCH, DC), (1, 0, 2))
    msg0 = _sc_msg(xc[0], xc[1], xc[2], xc[3], esrc, edst, zeros32)
    h1, h1c = _layer_tc(x, msg0, cnt, Ws0, Wn0, b0, ln_g0, ln_b0)

    msg1 = _sc_msg(h1c[0], h1c[1], h1c[2], h1c[3], esrc, edst, zeros32)
    h2, _ = _layer_tc(h1, msg1, cnt, Ws1, Wn1, b1, ln_g1, ln_b1)

    return _clf_tc(h2, W1, bc1, bn_g, bn_b, W2, bc2)


# speedup vs baseline: 1.3206x; 1.0092x over previous
import functools

import jax
import jax.numpy as jnp
from jax import lax
from jax.experimental import pallas as pl
from jax.experimental.pallas import tpu as pltpu
from jax.experimental.pallas import tpu_sc as plsc

N = 50000
R = 7
E = 80000
D = 128
H = 128
HC = 64
OUT = 2

NC = 2
NS = 16
NCH = 4
DC = 32
CNTW = 16

EP = E // NS
BATCH = 128
NBATCH = 40
EPP = NBATCH * BATCH
DUMP = 50040

NP = 50048
RPT = NP // NS
PIECE = 184
NPIECE = RPT // PIECE

_mesh = plsc.VectorSubcoreMesh(
    core_axis_name="c", subcore_axis_name="s", num_cores=NC, num_subcores=NS)


@functools.partial(
    pl.kernel,
    out_type=jax.ShapeDtypeStruct((R, NP, CNTW), jnp.float32),
    mesh=_mesh,
    compiler_params=pltpu.CompilerParams(use_tc_tiling_on_sc=False),
    scratch_types=dict(
        acc=pltpu.VMEM_SHARED((NP, CNTW), jnp.float32),
        ones=pltpu.VMEM((BATCH, CNTW), jnp.float32),
        didx=pltpu.VMEM((BATCH,), jnp.int32),
        zbuf=pltpu.VMEM((PIECE, CNTW), jnp.float32),
        wbuf=pltpu.VMEM((PIECE, CNTW), jnp.float32),
    ),
)
def _sc_counts(edst_hbm, zeros_hbm, ones_hbm, cnt_hbm, acc, ones, didx,
               zbuf, wbuf):
    cid = lax.axis_index("c")
    sid = lax.axis_index("s")
    pltpu.sync_copy(ones_hbm, ones)
    pltpu.sync_copy(zeros_hbm, zbuf)
    row0 = sid * RPT
    for r in range(R):
        mine = cid == (r % NC)

        @pl.when(mine)
        def _():
            def zstep(p, _):
                pltpu.sync_copy(zbuf, acc.at[pl.ds(row0 + p * PIECE, PIECE)])
                return 0

            lax.fori_loop(0, NPIECE, zstep, 0)

        plsc.subcore_barrier()

        @pl.when(mine)
        def _():
            base = (r * NS + sid) * EPP

            def bstep(b, _):
                pltpu.sync_copy(edst_hbm.at[pl.ds(base + b * BATCH, BATCH)],
                                didx)
                pltpu.sync_copy(ones, acc.at[didx], add=True)
                return 0

            lax.fori_loop(0, NBATCH, bstep, 0)

        plsc.subcore_barrier()

        @pl.when(mine)
        def _():
            def wstep(p, _):
                o = row0 + p * PIECE
                pltpu.sync_copy(acc.at[pl.ds(o, PIECE)], wbuf)
                pltpu.sync_copy(wbuf, cnt_hbm.at[r, pl.ds(o, PIECE)])
                return 0

            lax.fori_loop(0, NPIECE, wstep, 0)

        plsc.subcore_barrier()


@functools.partial(
    pl.kernel,
    out_type=jax.ShapeDtypeStruct((R, NCH, NP, DC), jnp.float32),
    mesh=_mesh,
    compiler_params=pltpu.CompilerParams(use_tc_tiling_on_sc=False),
    scratch_types=dict(
        acc=pltpu.VMEM_SHARED((NP, DC), jnp.float32),
        sidx=pltpu.VMEM((BATCH,), jnp.int32),
        didx=pltpu.VMEM((BATCH,), jnp.int32),
        rows=pltpu.VMEM((BATCH, DC), jnp.float32),
        sidx2=pltpu.VMEM((BATCH,), jnp.int32),
        didx2=pltpu.VMEM((BATCH,), jnp.int32),
        rows2=pltpu.VMEM((BATCH, DC), jnp.float32),
        zbuf=pltpu.VMEM((PIECE, DC), jnp.float32),
        wbuf=pltpu.VMEM((PIECE, DC), jnp.float32),
        wbuf2=pltpu.VMEM((PIECE, DC), jnp.float32),
        sem=pltpu.SemaphoreType.DMA,
        sem2=pltpu.SemaphoreType.DMA,
    ),
)
def _sc_msg(h0, h1, h2, h3, esrc_hbm, edst_hbm, zeros_hbm, msg_hbm, acc,
            sidx, didx, rows, sidx2, didx2, rows2, zbuf, wbuf, wbuf2, sem,
            sem2):
    cid = lax.axis_index("c")
    sid = lax.axis_index("s")
    pltpu.sync_copy(zeros_hbm, zbuf)
    row0 = sid * RPT

    def zstep(p, _):
        pltpu.sync_copy(zbuf, acc.at[pl.ds(row0 + p * PIECE, PIECE)])
        return 0

    lax.fori_loop(0, NPIECE, zstep, 0)
    plsc.subcore_barrier()

    hcs = [h0, h1, h2, h3]
    for r in range(R):
        for c in range(NCH):
            hc = hcs[c]
            t = r * NCH + c

            mine = cid == (t % NC)

            @pl.when(mine)
            def _():
                base = (r * NS + sid) * EPP

                def bstep(i, _):
                    o = base + (2 * i) * BATCH
                    o2 = o + BATCH
                    pltpu.sync_copy(esrc_hbm.at[pl.ds(o, BATCH)], sidx)
                    pltpu.sync_copy(edst_hbm.at[pl.ds(o, BATCH)], didx)
                    g = pltpu.async_copy(hc.at[sidx], rows, sem)
                    pltpu.sync_copy(esrc_hbm.at[pl.ds(o2, BATCH)], sidx2)
                    pltpu.sync_copy(edst_hbm.at[pl.ds(o2, BATCH)], didx2)
                    g2 = pltpu.async_copy(hc.at[sidx2], rows2, sem2)
                    g.wait()
                    pltpu.sync_copy(rows, acc.at[didx], add=True)
                    g2.wait()
                    pltpu.sync_copy(rows2, acc.at[didx2], add=True)
                    return 0

                lax.fori_loop(0, NBATCH // 2, bstep, 0)

            plsc.subcore_barrier()

            @pl.when(mine)
            def _():
                def wstep(i, _):
                    o = row0 + (2 * i) * PIECE
                    o2 = o + PIECE
                    pltpu.sync_copy(acc.at[pl.ds(o, PIECE)], wbuf)
                    w = pltpu.async_copy(
                        wbuf, msg_hbm.at[r, c, pl.ds(o, PIECE)], sem)
                    pltpu.sync_copy(zbuf, acc.at[pl.ds(o, PIECE)])
                    pltpu.sync_copy(acc.at[pl.ds(o2, PIECE)], wbuf2)
                    w2 = pltpu.async_copy(
                        wbuf2, msg_hbm.at[r, c, pl.ds(o2, PIECE)], sem2)
                    pltpu.sync_copy(zbuf, acc.at[pl.ds(o2, PIECE)])
                    w.wait()
                    w2.wait()
                    return 0

                lax.fori_loop(0, NPIECE // 2, wstep, 0)
                o = row0 + (NPIECE - 1) * PIECE
                pltpu.sync_copy(acc.at[pl.ds(o, PIECE)], wbuf)
                w = pltpu.async_copy(
                    wbuf, msg_hbm.at[r, c, pl.ds(o, PIECE)], sem)
                pltpu.sync_copy(zbuf, acc.at[pl.ds(o, PIECE)])
                w.wait()

            plsc.subcore_barrier()


BN = 1000


def _layer_body(h_ref, msg_ref, cnt_ref, Ws_ref, Wn_ref, b_ref, g_ref, be_ref,
                out_ref, outc_ref):
    h = h_ref[...]
    Ws_avg = jnp.mean(Ws_ref[...], axis=0)
    acc = jnp.dot(h, Ws_avg, preferred_element_type=jnp.float32)
    acc = acc + jnp.mean(b_ref[...], axis=0)[None, :]
    for r in range(R):
        inv = 1.0 / (R * jnp.maximum(cnt_ref[r, :, 0], 1.0))
        m = jnp.concatenate([msg_ref[r, c] for c in range(NCH)], axis=-1)
        acc = acc + jnp.dot(m * inv[:, None], Wn_ref[r],
                            preferred_element_type=jnp.float32)
    mu = jnp.mean(acc, axis=-1, keepdims=True)
    var = jnp.mean((acc - mu) * (acc - mu), axis=-1, keepdims=True)
    hn = (acc - mu) * lax.rsqrt(var + 1e-5)
    hn = hn * g_ref[...] + be_ref[...]
    hn = jnp.maximum(hn, 0.0)
    out_ref[...] = hn
    for c in range(NCH):
        outc_ref[c] = hn[:, c * DC:(c + 1) * DC]


def _layer_tc(h, msg, cnt, Ws, Wn, b, g, be):
    g2 = g.reshape(1, H)
    be2 = be.reshape(1, H)
    return pl.pallas_call(
        _layer_body,
        grid=(N // BN,),
        in_specs=[
            pl.BlockSpec((BN, H), lambda i: (i, 0)),
            pl.BlockSpec((R, NCH, BN, DC), lambda i: (0, 0, i, 0)),
            pl.BlockSpec((R, BN, CNTW), lambda i: (0, i, 0)),
            pl.BlockSpec((R, D, H), lambda i: (0, 0, 0)),
            pl.BlockSpec((R, D, H), lambda i: (0, 0, 0)),
            pl.BlockSpec((R, H), lambda i: (0, 0)),
            pl.BlockSpec((1, H), lambda i: (0, 0)),
            pl.BlockSpec((1, H), lambda i: (0, 0)),
        ],
        out_specs=[
            pl.BlockSpec((BN, H), lambda i: (i, 0)),
            pl.BlockSpec((NCH, BN, DC), lambda i: (0, i, 0)),
        ],
        out_shape=[
            jax.ShapeDtypeStruct((N, H), jnp.float32),
            jax.ShapeDtypeStruct((NCH, N, DC), jnp.float32),
        ],
    )(h, msg, cnt, Ws, Wn, b, g2, be2)


def _clf_body(h_ref, W1_ref, b1_ref, g_ref, be_ref, W2_ref, b2_ref, out_ref):
    t = jnp.dot(h_ref[...], W1_ref[...], preferred_element_type=jnp.float32)
    t = t + b1_ref[...]
    t = jnp.maximum(t, 0.0)
    t = t / jnp.sqrt(1.0 + 1e-5) * g_ref[...] + be_ref[...]
    out_ref[...] = jnp.dot(t, W2_ref[...],
                           preferred_element_type=jnp.float32) + b2_ref[...]


def _clf_tc(h, W1, b1, g, be, W2, b2):
    return pl.pallas_call(
        _clf_body,
        grid=(N // BN,),
        in_specs=[
            pl.BlockSpec((BN, H), lambda i: (i, 0)),
            pl.BlockSpec((H, HC), lambda i: (0, 0)),
            pl.BlockSpec((1, HC), lambda i: (0, 0)),
            pl.BlockSpec((1, HC), lambda i: (0, 0)),
            pl.BlockSpec((1, HC), lambda i: (0, 0)),
            pl.BlockSpec((HC, OUT), lambda i: (0, 0)),
            pl.BlockSpec((1, OUT), lambda i: (0, 0)),
        ],
        out_specs=pl.BlockSpec((BN, OUT), lambda i: (i, 0)),
        out_shape=jax.ShapeDtypeStruct((N, OUT), jnp.float32),
    )(h, W1, b1.reshape(1, HC), g.reshape(1, HC), be.reshape(1, HC),
      W2, b2.reshape(1, OUT))


def kernel(x, edge_index, Ws0, Wn0, b0, ln_g0, ln_b0, Ws1, Wn1, b1, ln_g1,
           ln_b1, W1, bc1, bn_g, bn_b, W2, bc2):
    zeros32 = jnp.zeros((PIECE, DC), jnp.float32)
    zeros16 = jnp.zeros((PIECE, CNTW), jnp.float32)
    ones16 = jnp.ones((BATCH, CNTW), jnp.float32)

    et = edge_index.reshape(R, 2, NS, EP)
    pad = [(0, 0)] * 2 + [(0, EPP - EP)]
    esrc = jnp.pad(et[:, 0], pad, constant_values=0).reshape(-1)
    edst = jnp.pad(et[:, 1], pad, constant_values=DUMP).reshape(-1)

    cnt = _sc_counts(edst, zeros16, ones16)

    xc = jnp.transpose(x.reshape(N, NCH, DC), (1, 0, 2))
    msg0 = _sc_msg(xc[0], xc[1], xc[2], xc[3], esrc, edst, zeros32)
    h1, h1c = _layer_tc(x, msg0, cnt, Ws0, Wn0, b0, ln_g0, ln_b0)

    msg1 = _sc_msg(h1c[0], h1c[1], h1c[2], h1c[3], esrc, edst, zeros32)
    h2, _ = _layer_tc(h1, msg1, cnt, Ws1, Wn1, b1, ln_g1, ln_b1)

    return _clf_tc(h2, W1, bc1, bn_g, bn_b, W2, bc2)
